# baseline (speedup 1.0000x reference)
import functools

import jax
import jax.numpy as jnp
from jax import lax
from jax.experimental import pallas as pl
from jax.experimental.pallas import tpu as pltpu
from jax.experimental.pallas import tpu_sc as plsc


def _make_gather(batch, seq, vocab, dim, num_cores, num_subcores):
    num_tokens = batch * seq
    nw = num_cores * num_subcores
    per_w = num_tokens // nw
    w_per_row = seq // per_w
    chunk = 32
    nchunk = per_w // chunk

    mesh = plsc.VectorSubcoreMesh(core_axis_name="c", subcore_axis_name="s")

    @functools.partial(
        pl.kernel,
        mesh=mesh,
        out_type=jax.ShapeDtypeStruct((num_tokens, dim), jnp.float32),
        scratch_types=[
            pltpu.VMEM((per_w,), jnp.int32),
            pltpu.VMEM((chunk, dim), jnp.float32),
            pltpu.VMEM((chunk, dim), jnp.float32),
            pltpu.SemaphoreType.DMA,
            pltpu.SemaphoreType.DMA,
            pltpu.SemaphoreType.DMA,
            pltpu.SemaphoreType.DMA,
        ],
    )
    def gather_k(idx_hbm, table_hbm, out_hbm, idx_v, buf0, buf1,
                 gs0, gs1, os0, os1):
        wid = lax.axis_index("s") * num_cores + lax.axis_index("c")
        base = wid * per_w
        bufs, gsems, osems = (buf0, buf1), (gs0, gs1), (os0, os1)
        pltpu.sync_copy(
            idx_hbm.at[wid // w_per_row,
                       pl.ds((wid % w_per_row) * per_w, per_w)],
            idx_v)

        def start_gather(ch):
            return pltpu.async_copy(
                table_hbm.at[idx_v.at[pl.ds(ch * chunk, chunk)]],
                bufs[ch % 2], gsems[ch % 2])

        def start_store(ch):
            return pltpu.async_copy(
                bufs[ch % 2], out_hbm.at[pl.ds(base + ch * chunk, chunk)],
                osems[ch % 2])

        gathers = [start_gather(0)]
        stores = [None] * nchunk
        for ch in range(nchunk):
            if ch + 1 < nchunk:
                if ch >= 1:
                    stores[ch - 1].wait()
                gathers.append(start_gather(ch + 1))
            gathers[ch].wait()
            stores[ch] = start_store(ch)
        if nchunk >= 2:
            stores[nchunk - 2].wait()
        stores[nchunk - 1].wait()

    return gather_k


def kernel(input_ids, embedding):
    batch, seq = input_ids.shape
    vocab, dim = embedding.shape
    num_tokens = batch * seq

    info = plsc.get_sparse_core_info()
    gather_k = _make_gather(
        batch, seq, vocab, dim, info.num_cores, info.num_subcores
    )
    out = gather_k(input_ids.astype(jnp.int32), embedding)
    return out.reshape(batch, seq, dim)


# speedup vs baseline: 1.0075x; 1.0075x over previous
import functools

import jax
import jax.numpy as jnp
from jax import lax
from jax.experimental import pallas as pl
from jax.experimental.pallas import tpu as pltpu
from jax.experimental.pallas import tpu_sc as plsc


def _make_gather(batch, seq, vocab, dim, num_cores, num_subcores):
    num_tokens = batch * seq
    nw = num_cores * num_subcores
    per_w = num_tokens // nw
    w_per_row = seq // per_w
    chunk = 64
    nchunk = per_w // chunk

    mesh = plsc.VectorSubcoreMesh(core_axis_name="c", subcore_axis_name="s")

    @functools.partial(
        pl.kernel,
        mesh=mesh,
        out_type=jax.ShapeDtypeStruct((num_tokens, dim), jnp.float32),
        scratch_types=[
            pltpu.VMEM((per_w,), jnp.int32),
            pltpu.VMEM((chunk, dim), jnp.float32),
            pltpu.SemaphoreType.DMA,
            pltpu.SemaphoreType.DMA,
        ],
    )
    def gather_k(idx_hbm, table_hbm, out_hbm, idx_v, buf, gs, os):
        wid = lax.axis_index("s") * num_cores + lax.axis_index("c")
        base = wid * per_w
        pltpu.sync_copy(
            idx_hbm.at[wid // w_per_row,
                       pl.ds((wid % w_per_row) * per_w, per_w)],
            idx_v)

        def body(ch, carry):
            off = pl.multiple_of(ch * chunk, 8)
            pltpu.async_copy(
                table_hbm.at[idx_v.at[pl.ds(off, chunk)]], buf, gs).wait()
            pltpu.async_copy(
                buf, out_hbm.at[pl.ds(base + off, chunk)], os).wait()
            return carry

        lax.fori_loop(0, nchunk, body, 0)

    return gather_k


def kernel(input_ids, embedding):
    batch, seq = input_ids.shape
    vocab, dim = embedding.shape
    num_tokens = batch * seq

    info = plsc.get_sparse_core_info()
    gather_k = _make_gather(
        batch, seq, vocab, dim, info.num_cores, info.num_subcores
    )
    out = gather_k(input_ids.astype(jnp.int32), embedding)
    return out.reshape(batch, seq, dim)
